# baseline (speedup 1.0000x reference)
import functools

import jax
import jax.numpy as jnp
from jax import lax
from jax.experimental import pallas as pl
from jax.experimental.pallas import tpu as pltpu
from jax.experimental.pallas import tpu_sc as plsc

B = 256
NK = 17
NCH = 56
NCELLS = 8400
G_TOTAL = B * NK
NW = 32
KP_PER_W = 144
G_PAD = NW * KP_PER_W
CHUNKS = KP_PER_W // 16
NIDX = KP_PER_W * 9
GCHUNK = 72
NGATHER = NIDX // GCHUNK

SS = (80, 40, 20)
OFFS = (0, 6400, 8000)


def _sc_gather_body(src, gtx, gty, out, xv, yv, idxv, rowsv, sem):
    wid = lax.axis_index("s") * 2 + lax.axis_index("c")
    base = wid * KP_PER_W
    pltpu.sync_copy(gtx.at[pl.ds(base, KP_PER_W)], xv)
    pltpu.sync_copy(gty.at[pl.ds(base, KP_PER_W)], yv)
    lane = lax.iota(jnp.int32, 16)
    for i in range(CHUNKS):
        x = xv[pl.ds(i * 16, 16)]
        y = yv[pl.ds(i * 16, 16)]
        g = jnp.minimum(base + i * 16 + lane, G_TOTAL - 1)
        b = lax.shift_right_logical(g * 7711, 17)
        k = g - b * NK
        common = b * (NCH * NCELLS) + (5 + 3 * k) * NCELLS
        for si in range(3):
            ss = SS[si]
            inv = float(ss) / 640.0
            ax = jnp.minimum((x * inv).astype(jnp.int32), ss - 1)
            ay = jnp.minimum((y * inv).astype(jnp.int32), ss - 1)
            cell = ax * ss + ay + OFFS[si]
            for ci in range(3):
                j = si * 3 + ci
                idxv[pl.ds(j * KP_PER_W + i * 16, 16)] = (
                    common + ci * NCELLS + cell)
    copies = []
    for t in range(NGATHER):
        sl = pl.ds(t * GCHUNK, GCHUNK)
        copies.append(pltpu.async_copy(src.at[idxv.at[sl]], rowsv.at[sl], sem))
    for cp in copies:
        cp.wait()
    for j in range(9):
        pltpu.sync_copy(rowsv.at[pl.ds(j * KP_PER_W, KP_PER_W)],
                        out.at[pl.ds(j * G_PAD + base, KP_PER_W)])


_sc_gather = pl.kernel(
    _sc_gather_body,
    out_type=jax.ShapeDtypeStruct((9 * G_PAD,), jnp.float32),
    mesh=plsc.VectorSubcoreMesh(core_axis_name="c", subcore_axis_name="s"),
    scratch_types=[
        pltpu.VMEM((KP_PER_W,), jnp.float32),
        pltpu.VMEM((KP_PER_W,), jnp.float32),
        pltpu.VMEM((NIDX,), jnp.int32),
        pltpu.VMEM((NIDX,), jnp.float32),
        pltpu.SemaphoreType.DMA,
    ],
)


def _tc_loss_body(g_ref, x_ref, y_ref, v_ref, out_ref):
    x = x_ref[...]
    y = y_ref[...]
    t = v_ref[...]
    mask = (t > 0.0).astype(jnp.float32)
    denom = 2.0 * jnp.sum(mask) + 1e-6
    total = jnp.float32(0.0)
    for si in range(3):
        px = g_ref[3 * si + 0]
        py = g_ref[3 * si + 1]
        p = g_ref[3 * si + 2]
        logp = jnp.maximum(jnp.log(p), -100.0)
        log1mp = jnp.maximum(jnp.log(1.0 - p), -100.0)
        total = total + jnp.sum(-(t * logp + (1.0 - t) * log1mp)) / G_TOTAL
        d2 = ((px - x) ** 2 + (py - y) ** 2) * mask
        total = total + jnp.sum(d2) / denom
    out_ref[0, 0] = total


_tc_loss = pl.pallas_call(
    _tc_loss_body,
    out_shape=jax.ShapeDtypeStruct((1, 1), jnp.float32),
    in_specs=[pl.BlockSpec(memory_space=pltpu.VMEM)] * 4,
    out_specs=pl.BlockSpec(memory_space=pltpu.SMEM),
)


@jax.jit
def kernel(output, gt_keypoints, keypoint_visibility):
    src = output.reshape(-1)
    gtx = gt_keypoints[:, :, 0]
    gty = gt_keypoints[:, :, 1]
    pad = G_PAD - G_TOTAL
    gathered = _sc_gather(src,
                          jnp.pad(gtx.reshape(-1), (0, pad)),
                          jnp.pad(gty.reshape(-1), (0, pad)))
    g9 = gathered.reshape(9, G_PAD)[:, :G_TOTAL].reshape(9, B, NK)
    loss = _tc_loss(g9, gtx, gty, keypoint_visibility)
    return loss[0, 0]


# speedup vs baseline: 28.2243x; 28.2243x over previous
import functools

import jax
import jax.numpy as jnp
from jax import lax
from jax.experimental import pallas as pl
from jax.experimental.pallas import tpu as pltpu
from jax.experimental.pallas import tpu_sc as plsc

B = 256
NK = 17
NCH = 56
NCELLS = 8400
G_TOTAL = B * NK
NW = 32
KP_PER_W = 144
G_PAD = NW * KP_PER_W
CHUNKS = KP_PER_W // 16
NIDX = KP_PER_W * 9
GCHUNK = 72
NGATHER = NIDX // GCHUNK

SS = (80, 40, 20)
OFFS = (0, 6400, 8000)


def _sc_gather_body(src, gtx, gty, out, xv, yv, idxv, rowsv, sem):
    wid = lax.axis_index("s") * 2 + lax.axis_index("c")
    base = wid * KP_PER_W
    pltpu.sync_copy(gtx.at[pl.ds(base, KP_PER_W)], xv)
    pltpu.sync_copy(gty.at[pl.ds(base, KP_PER_W)], yv)
    lane = lax.iota(jnp.int32, 16)
    for i in range(CHUNKS):
        x = xv[pl.ds(i * 16, 16)]
        y = yv[pl.ds(i * 16, 16)]
        g = jnp.minimum(base + i * 16 + lane, G_TOTAL - 1)
        b = lax.shift_right_logical(g * 7711, 17)
        k = g - b * NK
        bterm = (lax.shift_right_logical(b, 7) * 1024
                 + jnp.bitwise_and(b, 127))
        ch0 = 5 + 3 * k
        for si in range(3):
            ss = SS[si]
            inv = float(ss) / 640.0
            ax = jnp.minimum((x * inv).astype(jnp.int32), ss - 1)
            ay = jnp.minimum((y * inv).astype(jnp.int32), ss - 1)
            cell = ax * ss + ay + OFFS[si]
            cterm = (lax.shift_right_logical(cell, 3) * 2048
                     + jnp.bitwise_and(cell, 7) * 128 + bterm)
            for ci in range(3):
                j = si * 3 + ci
                idxv[pl.ds(j * KP_PER_W + i * 16, 16)] = (
                    (ch0 + ci) * 2150400 + cterm)
    copies = []
    for t in range(NGATHER):
        sl = pl.ds(t * GCHUNK, GCHUNK)
        copies.append(pltpu.async_copy(src.at[idxv.at[sl]], rowsv.at[sl], sem))
    for cp in copies:
        cp.wait()
    for j in range(9):
        pltpu.sync_copy(rowsv.at[pl.ds(j * KP_PER_W, KP_PER_W)],
                        out.at[pl.ds(j * G_PAD + base, KP_PER_W)])


_sc_gather = pl.kernel(
    _sc_gather_body,
    out_type=jax.ShapeDtypeStruct((9 * G_PAD,), jnp.float32),
    mesh=plsc.VectorSubcoreMesh(core_axis_name="c", subcore_axis_name="s"),
    scratch_types=[
        pltpu.VMEM((KP_PER_W,), jnp.float32),
        pltpu.VMEM((KP_PER_W,), jnp.float32),
        pltpu.VMEM((NIDX,), jnp.int32),
        pltpu.VMEM((NIDX,), jnp.float32),
        pltpu.SemaphoreType.DMA,
    ],
)


def _tc_loss_body(g_ref, x_ref, y_ref, v_ref, out_ref):
    x = x_ref[...]
    y = y_ref[...]
    t = v_ref[...]
    mask = (t > 0.0).astype(jnp.float32)
    denom = 2.0 * jnp.sum(mask) + 1e-6
    total = jnp.float32(0.0)
    for si in range(3):
        px = g_ref[3 * si + 0]
        py = g_ref[3 * si + 1]
        p = g_ref[3 * si + 2]
        logp = jnp.maximum(jnp.log(p), -100.0)
        log1mp = jnp.maximum(jnp.log(1.0 - p), -100.0)
        total = total + jnp.sum(-(t * logp + (1.0 - t) * log1mp)) / G_TOTAL
        d2 = ((px - x) ** 2 + (py - y) ** 2) * mask
        total = total + jnp.sum(d2) / denom
    out_ref[0, 0] = total


_tc_loss = pl.pallas_call(
    _tc_loss_body,
    out_shape=jax.ShapeDtypeStruct((1, 1), jnp.float32),
    in_specs=[pl.BlockSpec(memory_space=pltpu.VMEM)] * 4,
    out_specs=pl.BlockSpec(memory_space=pltpu.SMEM),
)


@jax.jit
def kernel(output, gt_keypoints, keypoint_visibility):
    src = (output.transpose(1, 2, 0)
           .reshape(58800, 8, 2, 128)
           .transpose(0, 2, 1, 3)
           .reshape(-1))
    gtx = gt_keypoints[:, :, 0]
    gty = gt_keypoints[:, :, 1]
    pad = G_PAD - G_TOTAL
    gathered = _sc_gather(src,
                          jnp.pad(gtx.reshape(-1), (0, pad)),
                          jnp.pad(gty.reshape(-1), (0, pad)))
    g9 = gathered.reshape(9, G_PAD)[:, :G_TOTAL].reshape(9, B, NK)
    loss = _tc_loss(g9, gtx, gty, keypoint_visibility)
    return loss[0, 0]
